# baseline (speedup 1.0000x reference)
import functools

import jax
import jax.numpy as jnp
from jax import lax
from jax.experimental import pallas as pl
from jax.experimental.pallas import tpu as pltpu
from jax.experimental.pallas import tpu_sc as plsc

N = 10000
E = 320000
D_IN = 128
H = 64
D_OUT = 128
L = 4
G = 16

W = 80
NPAD = 10240
NT = 10016
WS = 96
WT = 96
NW = 32
EW = 10240
EPAD = NW * EW
CH = 128
NCHUNK = EW // CH
BN = 2048
BE = 2048


def _silu(x):
    return x * jax.lax.logistic(x)


def _make_sc_gather_body(nch, cbase):
    ew = nch * CH

    def body(ta, tb, row2d, col2d, out, idxr, idxc,
             a0, a1, a2, a3, b0, b1, b2, b3, sem_g, sem_s):
        c = lax.axis_index("c")
        s = lax.axis_index("s")
        wid = s * 2 + c
        ab = (a0, a1, a2, a3)
        bb = (b0, b1, b2, b3)

        pltpu.sync_copy(row2d.at[pl.ds(cbase + wid * nch, nch)], idxr)
        pltpu.sync_copy(col2d.at[pl.ds(cbase + wid * nch, nch)], idxc)

        def fire(k, p):
            pltpu.async_copy(ta.at[idxr.at[k]], ab[p], sem_g)
            pltpu.async_copy(tb.at[idxc.at[k]], bb[p], sem_g)

        fire(0, 0)
        fire(1, 1)

        def addp(ap, bp):
            @plsc.parallel_loop(0, CH, unroll=4)
            def _(g):
                for i in range(3):
                    sl = pl.ds(i * 32, 32)
                    ap[g, sl] = ap[g, sl] + bp[g, sl]

        def quad(j, carry):
            for q in range(4):
                k = 4 * j + q
                base = wid * ew + k * CH
                dummy = out.at[pl.ds(base, CH)]
                pltpu.make_async_copy(dummy, a0, sem_g).wait()
                pltpu.make_async_copy(dummy, b0, sem_g).wait()

                @pl.when(k >= 2)
                def _():
                    pltpu.make_async_copy(a0, dummy, sem_s).wait()

                @pl.when(k + 2 < nch)
                def _():
                    fire(k + 2, (q + 2) % 4)

                addp(ab[q], bb[q])
                pltpu.async_copy(ab[q], out.at[pl.ds(base, CH)], sem_s)
            return carry

        lax.fori_loop(0, nch // 4, quad, 0)
        dummy = out.at[pl.ds(wid * ew, CH)]
        pltpu.make_async_copy(a0, dummy, sem_s).wait()
        pltpu.make_async_copy(a0, dummy, sem_s).wait()

    return body


def _sc_gather(ta, tb, row2d, col2d, nch, cbase):
    mesh = plsc.VectorSubcoreMesh(core_axis_name="c", subcore_axis_name="s")
    fn = pl.kernel(
        _make_sc_gather_body(nch, cbase),
        out_type=jax.ShapeDtypeStruct((NW * nch * CH, WT), jnp.bfloat16),
        mesh=mesh,
        compiler_params=pltpu.CompilerParams(use_tc_tiling_on_sc=False),
        scratch_types=[
            pltpu.VMEM((nch, CH), jnp.int32),
            pltpu.VMEM((nch, CH), jnp.int32),
            pltpu.VMEM((CH, WT), jnp.bfloat16),
            pltpu.VMEM((CH, WT), jnp.bfloat16),
            pltpu.VMEM((CH, WT), jnp.bfloat16),
            pltpu.VMEM((CH, WT), jnp.bfloat16),
            pltpu.VMEM((CH, WT), jnp.bfloat16),
            pltpu.VMEM((CH, WT), jnp.bfloat16),
            pltpu.VMEM((CH, WT), jnp.bfloat16),
            pltpu.VMEM((CH, WT), jnp.bfloat16),
            pltpu.SemaphoreType.DMA,
            pltpu.SemaphoreType.DMA,
        ],
    )
    return fn(ta, tb, row2d, col2d)


def _make_sc_scatter_body(nch, cbase):
    ew = nch * CH

    def body(feat, row2d, out0, out1, idx,
             f0, f1, f2, f3, f4, f5, acc, sem_l, sem_sc):
        c = lax.axis_index("c")
        s = lax.axis_index("s")
        wid = s * 2 + c
        fb = (f0, f1, f2, f3, f4, f5)

        pltpu.sync_copy(row2d.at[pl.ds(cbase + wid * nch, nch)], idx)

        zv = jnp.zeros((32,), jnp.bfloat16)

        def zrow(g, cc):
            for i in range(3):
                f0[g, pl.ds(i * 32, 32)] = zv
            return cc

        lax.fori_loop(0, CH, zrow, 0)

        def zcp(j, cc):
            pltpu.sync_copy(f0, acc.at[pl.ds(s * (NPAD // 16) + j * CH, CH)])
            return cc

        lax.fori_loop(0, (NPAD // 16) // CH, zcp, 0)
        plsc.subcore_barrier()

        def fire(k, p):
            pltpu.async_copy(feat.at[pl.ds(wid * ew + k * CH, CH)], fb[p],
                             sem_l)

        fire(0, 0)
        fire(1, 1)
        fire(2, 2)

        def step(k, q):
            dummy = feat.at[pl.ds(wid * ew + k * CH, CH)]
            pltpu.make_async_copy(dummy, f0, sem_l).wait()

            @pl.when(k >= 3)
            def _():
                pltpu.make_async_copy(dummy, f0, sem_sc).wait()

            @pl.when(k + 3 < nch)
            def _():
                fire(k + 3, (q + 3) % 6)

            pltpu.async_copy(fb[q], acc.at[idx.at[k]], sem_sc, add=True)

        def hexa(j, cc):
            for q in range(6):
                step(6 * j + q, q)
            return cc

        lax.fori_loop(0, nch // 6, hexa, 0)
        for q in range(nch - (nch // 6) * 6):
            k = (nch // 6) * 6 + q
            step(k, k % 6)
        dummy = feat.at[pl.ds(wid * ew, CH)]
        for _ in range(3):
            pltpu.make_async_copy(dummy, f0, sem_sc).wait()
        plsc.subcore_barrier()

        def cpout(j, cc):
            off = s * (NPAD // 16) + j * CH

            @pl.when(c == 0)
            def _():
                pltpu.sync_copy(acc.at[pl.ds(off, CH)],
                                out0.at[pl.ds(off, CH)])

            @pl.when(c == 1)
            def _():
                pltpu.sync_copy(acc.at[pl.ds(off, CH)],
                                out1.at[pl.ds(off, CH)])

            return cc

        lax.fori_loop(0, (NPAD // 16) // CH, cpout, 0)

    return body


def _sc_scatter(feat, row2d, nch, cbase):
    mesh = plsc.VectorSubcoreMesh(core_axis_name="c", subcore_axis_name="s")
    fn = pl.kernel(
        _make_sc_scatter_body(nch, cbase),
        out_type=(
            jax.ShapeDtypeStruct((NPAD, WS), jnp.bfloat16),
            jax.ShapeDtypeStruct((NPAD, WS), jnp.bfloat16),
        ),
        mesh=mesh,
        compiler_params=pltpu.CompilerParams(use_tc_tiling_on_sc=False),
        scratch_types=[
            pltpu.VMEM((nch, CH), jnp.int32),
            pltpu.VMEM((CH, WS), jnp.bfloat16),
            pltpu.VMEM((CH, WS), jnp.bfloat16),
            pltpu.VMEM((CH, WS), jnp.bfloat16),
            pltpu.VMEM((CH, WS), jnp.bfloat16),
            pltpu.VMEM((CH, WS), jnp.bfloat16),
            pltpu.VMEM((CH, WS), jnp.bfloat16),
            pltpu.VMEM_SHARED((NPAD, WS), jnp.bfloat16),
            pltpu.SemaphoreType.DMA,
            pltpu.SemaphoreType.DMA,
        ],
    )
    return fn(feat, row2d)


def _tc_edge_body(x_ref, w1r, ew2, eb2, cw1, cb1, cw2r, o_ref):
    x = x_ref[...].astype(jnp.float32)
    hi = x[:, 64:80]
    radial = jnp.sum(hi * hi, axis=1, keepdims=True)
    m = _silu(x[:, 0:64] + radial * w1r[...])
    f = _silu(jnp.dot(m, ew2[...], preferred_element_type=jnp.float32) + eb2[...])
    c2 = _silu(jnp.dot(f, cw1[...], preferred_element_type=jnp.float32) + cb1[...])
    phi = jnp.sum(c2 * cw2r[...], axis=1, keepdims=True)
    e3 = (lax.broadcasted_iota(jnp.int32, (1, 16), 1) == 3).astype(jnp.float32)
    o_ref[...] = jnp.concatenate(
        [f, hi * phi + e3, jnp.zeros((BE, WS - W), jnp.float32)],
        axis=1).astype(jnp.bfloat16)


def _tc_edge(mpre, w1r, ew2, eb2, cw1, cb1, cw2r):
    epart = mpre.shape[0]
    full = lambda shape: pl.BlockSpec(shape, lambda i: (0,) * len(shape))
    return pl.pallas_call(
        _tc_edge_body,
        grid=(epart // BE,),
        in_specs=[
            pl.BlockSpec((BE, WT), lambda i: (i, 0)),
            full((1, H)), full((H, H)), full((1, H)),
            full((H, H)), full((1, H)), full((1, H)),
        ],
        out_specs=pl.BlockSpec((BE, WS), lambda i: (i, 0)),
        out_shape=jax.ShapeDtypeStruct((epart, WS), jnp.bfloat16),
    )(mpre, w1r, ew2, eb2, cw1, cb1, cw2r)


def _tc_init_body(h_ref, cd_ref, win, bin_, ew1a, ew1b, eb1,
                  h1_ref, ta_ref, tb_ref):
    h1 = jnp.dot(h_ref[...], win[...], preferred_element_type=jnp.float32) + bin_[...]
    h1_ref[...] = h1
    chi = cd_ref[...]
    z16 = jnp.zeros((BN, WT - W), jnp.float32)
    fa = jnp.dot(h1, ew1a[...], preferred_element_type=jnp.float32) + eb1[...]
    fb = jnp.dot(h1, ew1b[...], preferred_element_type=jnp.float32)
    ta_ref[...] = jnp.concatenate([fa, chi, z16], axis=1).astype(jnp.bfloat16)
    tb_ref[...] = jnp.concatenate([fb, -chi, z16], axis=1).astype(jnp.bfloat16)


def _tc_init(hpad, cpad16, win, bin_, ew1a, ew1b, eb1):
    full = lambda shape: pl.BlockSpec(shape, lambda i: (0,) * len(shape))
    return pl.pallas_call(
        _tc_init_body,
        grid=(NPAD // BN,),
        in_specs=[
            pl.BlockSpec((BN, D_IN), lambda i: (i, 0)),
            pl.BlockSpec((BN, 16), lambda i: (i, 0)),
            full((D_IN, H)), full((1, H)),
            full((H, H)), full((H, H)), full((1, H)),
        ],
        out_specs=[
            pl.BlockSpec((BN, H), lambda i: (i, 0)),
            pl.BlockSpec((BN, WT), lambda i: (i, 0)),
            pl.BlockSpec((BN, WT), lambda i: (i, 0)),
        ],
        out_shape=[
            jax.ShapeDtypeStruct((NPAD, H), jnp.float32),
            jax.ShapeDtypeStruct((NPAD, WT), jnp.bfloat16),
            jax.ShapeDtypeStruct((NPAD, WT), jnp.bfloat16),
        ],
    )(hpad, cpad16, win, bin_, ew1a, ew1b, eb1)


def _tc_node_body(h_ref, cd_ref, p0_ref, p1_ref, p2_ref, p3_ref,
                  nw1a, nw1b, nb1, nw2, nb2,
                  ew1a, ew1b, eb1, h_out, cd_out, ta_ref, tb_ref):
    h = h_ref[...]
    f32 = jnp.float32
    s = ((p0_ref[...].astype(f32) + p1_ref[...].astype(f32))
         + (p2_ref[...].astype(f32) + p3_ref[...].astype(f32)))
    agg = s[:, 0:64]
    hi = s[:, 64:80]
    cnt = jnp.maximum(hi[:, 3:4], 1.0)
    chi = cd_ref[...]
    mask3 = (lax.broadcasted_iota(jnp.int32, (1, 16), 1) < 3).astype(jnp.float32)
    chi_new = chi + (hi / cnt) * mask3
    hm = _silu(jnp.dot(h, nw1a[...], preferred_element_type=jnp.float32)
               + jnp.dot(agg, nw1b[...], preferred_element_type=jnp.float32)
               + nb1[...])
    hn = h + jnp.dot(hm, nw2[...], preferred_element_type=jnp.float32) + nb2[...]
    h_out[...] = hn
    cd_out[...] = chi_new
    z16 = jnp.zeros((BN, WT - W), jnp.float32)
    fa = jnp.dot(hn, ew1a[...], preferred_element_type=jnp.float32) + eb1[...]
    fb = jnp.dot(hn, ew1b[...], preferred_element_type=jnp.float32)
    ta_ref[...] = jnp.concatenate([fa, chi_new, z16], axis=1).astype(jnp.bfloat16)
    tb_ref[...] = jnp.concatenate([fb, -chi_new, z16], axis=1).astype(jnp.bfloat16)


def _tc_node(h, cd16, p0, p1, p2, p3,
             nw1a, nw1b, nb1, nw2, nb2, ew1a, ew1b, eb1):
    full = lambda shape: pl.BlockSpec(shape, lambda i: (0,) * len(shape))
    return pl.pallas_call(
        _tc_node_body,
        grid=(NPAD // BN,),
        in_specs=[
            pl.BlockSpec((BN, H), lambda i: (i, 0)),
            pl.BlockSpec((BN, 16), lambda i: (i, 0)),
            pl.BlockSpec((BN, WS), lambda i: (i, 0)),
            pl.BlockSpec((BN, WS), lambda i: (i, 0)),
            pl.BlockSpec((BN, WS), lambda i: (i, 0)),
            pl.BlockSpec((BN, WS), lambda i: (i, 0)),
            full((H, H)), full((H, H)), full((1, H)), full((H, H)), full((1, H)),
            full((H, H)), full((H, H)), full((1, H)),
        ],
        out_specs=[
            pl.BlockSpec((BN, H), lambda i: (i, 0)),
            pl.BlockSpec((BN, 16), lambda i: (i, 0)),
            pl.BlockSpec((BN, WT), lambda i: (i, 0)),
            pl.BlockSpec((BN, WT), lambda i: (i, 0)),
        ],
        out_shape=[
            jax.ShapeDtypeStruct((NPAD, H), jnp.float32),
            jax.ShapeDtypeStruct((NPAD, 16), jnp.float32),
            jax.ShapeDtypeStruct((NPAD, WT), jnp.bfloat16),
            jax.ShapeDtypeStruct((NPAD, WT), jnp.bfloat16),
        ],
    )(h, cd16, p0, p1, p2, p3, nw1a, nw1b, nb1, nw2, nb2, ew1a, ew1b, eb1)


def _tc_final_body(h_ref, p0_ref, p1_ref, p2_ref, p3_ref,
                   nw1a, nw1b, nb1, nw2, nb2,
                   wout, bout, b_ref, o_ref):
    i = pl.program_id(0)
    h = h_ref[...]
    f32 = jnp.float32
    s = ((p0_ref[...].astype(f32) + p1_ref[...].astype(f32))
         + (p2_ref[...].astype(f32) + p3_ref[...].astype(f32)))
    agg = s[:, 0:64]
    hm = _silu(jnp.dot(h, nw1a[...], preferred_element_type=jnp.float32)
               + jnp.dot(agg, nw1b[...], preferred_element_type=jnp.float32)
               + nb1[...])
    hn = h + jnp.dot(hm, nw2[...], preferred_element_type=jnp.float32) + nb2[...]
    hf = jnp.dot(hn, wout[...], preferred_element_type=jnp.float32) + bout[...]
    hext = jnp.concatenate([hf, jnp.ones((BN, 1), jnp.float32)], axis=1)
    b = b_ref[...][0]
    rows = lax.broadcasted_iota(jnp.int32, (G, BN), 0)
    oh = (rows == b).astype(jnp.float32)
    contrib = jnp.dot(oh, hext, preferred_element_type=jnp.float32)

    @pl.when(i == 0)
    def _():
        o_ref[...] = contrib

    @pl.when(i > 0)
    def _():
        o_ref[...] = o_ref[...] + contrib

    @pl.when(i == NPAD // BN - 1)
    def _():
        acc = o_ref[...]
        o_ref[...] = acc / jnp.maximum(acc[:, D_OUT:D_OUT + 1], 1.0)


def _tc_final(h, p0, p1, p2, p3,
              nw1a, nw1b, nb1, nw2, nb2, wout, bout, batch3d):
    full = lambda shape: pl.BlockSpec(shape, lambda i: (0,) * len(shape))
    return pl.pallas_call(
        _tc_final_body,
        grid=(NPAD // BN,),
        in_specs=[
            pl.BlockSpec((BN, H), lambda i: (i, 0)),
            pl.BlockSpec((BN, WS), lambda i: (i, 0)),
            pl.BlockSpec((BN, WS), lambda i: (i, 0)),
            pl.BlockSpec((BN, WS), lambda i: (i, 0)),
            pl.BlockSpec((BN, WS), lambda i: (i, 0)),
            full((H, H)), full((H, H)), full((1, H)), full((H, H)), full((1, H)),
            full((H, D_OUT)), full((1, D_OUT)),
            pl.BlockSpec((1, 1, BN), lambda i: (i, 0, 0)),
        ],
        out_specs=pl.BlockSpec((G, D_OUT + 1), lambda i: (0, 0)),
        out_shape=jax.ShapeDtypeStruct((G, D_OUT + 1), jnp.float32),
    )(h, p0, p1, p2, p3, nw1a, nw1b, nb1, nw2, nb2, wout, bout, batch3d)


@jax.jit
def _run(h, x, W_in, b_in, edge_w1, edge_b1, edge_w2, edge_b2,
         coord_w1, coord_b1, coord_w2, node_w1, node_b1, node_w2, node_b2,
         W_out, b_out, edge_index, batch):
    f32 = jnp.float32
    hpad = jnp.zeros((NPAD, D_IN), f32).at[:N].set(h)
    cpad16 = jnp.zeros((NPAD, 16), f32).at[:N, :3].set(x)
    rowp = jnp.full((EPAD,), N, jnp.int32).at[:E].set(edge_index[0])
    colp = jnp.full((EPAD,), N, jnp.int32).at[:E].set(edge_index[1])
    rowp = rowp.reshape(EPAD // CH, CH)
    colp = colp.reshape(EPAD // CH, CH)
    batch3d = jnp.full((NPAD,), G + 1, jnp.int32).at[:N].set(batch)
    batch3d = batch3d.reshape(NPAD // BN, 1, BN)

    def r1(v):
        return v.reshape(1, -1)

    h1, ta, tb = _tc_init(hpad, cpad16, W_in, r1(b_in),
                          edge_w1[0, 0:H], edge_w1[0, H:2 * H], r1(edge_b1[0]))
    hcur = h1
    nch_h = NCHUNK // 2
    cb1_ = NW * nch_h
    for l in range(L):
        ew = (r1(edge_w1[l, 2 * H]), edge_w2[l], r1(edge_b2[l]),
              coord_w1[l], r1(coord_b1[l]), r1(coord_w2[l, :, 0]))
        m0 = _sc_gather(ta, tb, rowp, colp, nch_h, 0)
        m1 = _sc_gather(ta, tb, rowp, colp, nch_h, cb1_)
        f0 = _tc_edge(m0, *ew)
        f1 = _tc_edge(m1, *ew)
        p0, p1 = _sc_scatter(f0, rowp, nch_h, 0)
        p2, p3 = _sc_scatter(f1, rowp, nch_h, cb1_)
        if l < L - 1:
            hcur, cpad16_n, ta, tb = _tc_node(
                hcur, cpad16, p0, p1, p2, p3,
                node_w1[l, 0:H], node_w1[l, H:2 * H], r1(node_b1[l]),
                node_w2[l], r1(node_b2[l]),
                edge_w1[l + 1, 0:H], edge_w1[l + 1, H:2 * H], r1(edge_b1[l + 1]))
            cpad16 = cpad16_n
        else:
            out = _tc_final(hcur, p0, p1, p2, p3,
                            node_w1[l, 0:H], node_w1[l, H:2 * H], r1(node_b1[l]),
                            node_w2[l], r1(node_b2[l]),
                            W_out, r1(b_out), batch3d)
    return out[:, :D_OUT]


def kernel(h, x, W_in, b_in, edge_w1, edge_b1, edge_w2, edge_b2,
           coord_w1, coord_b1, coord_w2, node_w1, node_b1, node_w2, node_b2,
           W_out, b_out, edge_index, batch):
    return _run(h, x, W_in, b_in, edge_w1, edge_b1, edge_w2, edge_b2,
                coord_w1, coord_b1, coord_w2, node_w1, node_b1, node_w2,
                node_b2, W_out, b_out, edge_index, batch)


# speedup vs baseline: 1.1296x; 1.1296x over previous
import functools

import jax
import jax.numpy as jnp
from jax import lax
from jax.experimental import pallas as pl
from jax.experimental.pallas import tpu as pltpu
from jax.experimental.pallas import tpu_sc as plsc

N = 10000
E = 320000
D_IN = 128
H = 64
D_OUT = 128
L = 4
G = 16

W = 80
NPAD = 10240
NT = 10016
WS = 96
WT = 96
NW = 32
EW = 10240
EPAD = NW * EW
CH = 128
NCHUNK = EW // CH
BN = 2048
BE = 2048


def _silu(x):
    return x * jax.lax.logistic(x)


def _make_sc_gather_body(nch, cbase):
    ew = nch * CH

    def body(ta, tb, row2d, col2d, out, idxr, idxc,
             a0, a1, a2, a3, b0, b1, b2, b3, sem_g, sem_s):
        c = lax.axis_index("c")
        s = lax.axis_index("s")
        wid = s * 2 + c
        ab = (a0, a1, a2, a3)
        bb = (b0, b1, b2, b3)

        pltpu.sync_copy(row2d.at[pl.ds(cbase + wid * nch, nch)], idxr)
        pltpu.sync_copy(col2d.at[pl.ds(cbase + wid * nch, nch)], idxc)

        def fire(k, p):
            pltpu.async_copy(ta.at[idxr.at[k]], ab[p], sem_g)
            pltpu.async_copy(tb.at[idxc.at[k]], bb[p], sem_g)

        fire(0, 0)
        fire(1, 1)

        def addp(ap, bp):
            @plsc.parallel_loop(0, CH, unroll=4)
            def _(g):
                for i in range(5):
                    sl = pl.ds(i * 16, 16)
                    ap[g, sl] = ap[g, sl] + bp[g, sl]

        def quad(j, carry):
            for q in range(4):
                k = 4 * j + q
                base = wid * ew + k * CH
                dummy = out.at[pl.ds(base, CH)]
                pltpu.make_async_copy(dummy, a0, sem_g).wait()
                pltpu.make_async_copy(dummy, b0, sem_g).wait()

                @pl.when(k >= 2)
                def _():
                    pltpu.make_async_copy(a0, dummy, sem_s).wait()

                @pl.when(k + 2 < nch)
                def _():
                    fire(k + 2, (q + 2) % 4)

                addp(ab[q], bb[q])
                pltpu.async_copy(ab[q], out.at[pl.ds(base, CH)], sem_s)
            return carry

        lax.fori_loop(0, nch // 4, quad, 0)
        dummy = out.at[pl.ds(wid * ew, CH)]
        pltpu.make_async_copy(a0, dummy, sem_s).wait()
        pltpu.make_async_copy(a0, dummy, sem_s).wait()

    return body


def _sc_gather(ta, tb, row2d, col2d, nch, cbase):
    mesh = plsc.VectorSubcoreMesh(core_axis_name="c", subcore_axis_name="s")
    fn = pl.kernel(
        _make_sc_gather_body(nch, cbase),
        out_type=jax.ShapeDtypeStruct((NW * nch * CH, W), jnp.float32),
        mesh=mesh,
        compiler_params=pltpu.CompilerParams(use_tc_tiling_on_sc=False),
        scratch_types=[
            pltpu.VMEM((nch, CH), jnp.int32),
            pltpu.VMEM((nch, CH), jnp.int32),
            pltpu.VMEM((CH, W), jnp.float32),
            pltpu.VMEM((CH, W), jnp.float32),
            pltpu.VMEM((CH, W), jnp.float32),
            pltpu.VMEM((CH, W), jnp.float32),
            pltpu.VMEM((CH, W), jnp.float32),
            pltpu.VMEM((CH, W), jnp.float32),
            pltpu.VMEM((CH, W), jnp.float32),
            pltpu.VMEM((CH, W), jnp.float32),
            pltpu.SemaphoreType.DMA,
            pltpu.SemaphoreType.DMA,
        ],
    )
    return fn(ta, tb, row2d, col2d)


def _make_sc_scatter_body(nch, cbase):
    ew = nch * CH

    def body(feat, row2d, o00, o01, o10, o11, idx,
             f0, f1, f2, f3, f4, f5, acc0, acc1, sem_l, sem_sc):
        c = lax.axis_index("c")
        s = lax.axis_index("s")
        wid = s * 2 + c
        fb = (f0, f1, f2, f3, f4, f5)

        pltpu.sync_copy(row2d.at[pl.ds(cbase + wid * nch, nch)], idx)

        zv = jnp.zeros((32,), jnp.bfloat16)

        def zrow(g, cc):
            for i in range(3):
                f0[g, pl.ds(i * 32, 32)] = zv
            return cc

        lax.fori_loop(0, CH, zrow, 0)

        def zcp(j, cc):
            sl = pl.ds(s * (NPAD // 16) + j * CH, CH)
            pltpu.async_copy(f0, acc0.at[sl], sem_l)
            pltpu.async_copy(f0, acc1.at[sl], sem_l)
            return cc

        lax.fori_loop(0, (NPAD // 16) // CH, zcp, 0)

        def zwait(j, cc):
            sl = pl.ds(s * (NPAD // 16), CH)
            pltpu.make_async_copy(f0, acc0.at[sl], sem_l).wait()
            pltpu.make_async_copy(f0, acc1.at[sl], sem_l).wait()
            return cc

        lax.fori_loop(0, (NPAD // 16) // CH, zwait, 0)
        plsc.subcore_barrier()

        def fire(k, p):
            pltpu.async_copy(feat.at[pl.ds(wid * ew + k * CH, CH)], fb[p],
                             sem_l)

        fire(0, 0)
        fire(1, 1)
        fire(2, 2)

        def step(k, q):
            dummy = feat.at[pl.ds(wid * ew + k * CH, CH)]
            pltpu.make_async_copy(dummy, f0, sem_l).wait()

            @pl.when(k >= 3)
            def _():
                pltpu.make_async_copy(dummy, f0, sem_sc).wait()

            @pl.when(k + 3 < nch)
            def _():
                fire(k + 3, (q + 3) % 6)

            @pl.when(s < 8)
            def _():
                pltpu.async_copy(fb[q], acc0.at[idx.at[k]], sem_sc, add=True)

            @pl.when(s >= 8)
            def _():
                pltpu.async_copy(fb[q], acc1.at[idx.at[k]], sem_sc, add=True)

        def hexa(j, cc):
            for q in range(6):
                step(6 * j + q, q)
            return cc

        lax.fori_loop(0, nch // 6, hexa, 0)
        for q in range(nch - (nch // 6) * 6):
            k = (nch // 6) * 6 + q
            step(k, k % 6)
        dummy = feat.at[pl.ds(wid * ew, CH)]
        for _ in range(3):
            pltpu.make_async_copy(dummy, f0, sem_sc).wait()
        plsc.subcore_barrier()

        def cpout(j, cc):
            off = pl.ds(s * (NPAD // 16) + j * CH, CH)

            @pl.when(c == 0)
            def _():
                pltpu.async_copy(acc0.at[off], o00.at[off], sem_l)
                pltpu.async_copy(acc1.at[off], o01.at[off], sem_l)

            @pl.when(c == 1)
            def _():
                pltpu.async_copy(acc0.at[off], o10.at[off], sem_l)
                pltpu.async_copy(acc1.at[off], o11.at[off], sem_l)

            return cc

        lax.fori_loop(0, (NPAD // 16) // CH, cpout, 0)

        def cpwait(j, cc):
            off = pl.ds(s * (NPAD // 16), CH)
            pltpu.make_async_copy(acc0.at[off], o00.at[off], sem_l).wait()
            pltpu.make_async_copy(acc0.at[off], o00.at[off], sem_l).wait()
            return cc

        lax.fori_loop(0, (NPAD // 16) // CH, cpwait, 0)

    return body


def _sc_scatter(feat, row2d, nch, cbase):
    mesh = plsc.VectorSubcoreMesh(core_axis_name="c", subcore_axis_name="s")
    fn = pl.kernel(
        _make_sc_scatter_body(nch, cbase),
        out_type=(
            jax.ShapeDtypeStruct((NPAD, WS), jnp.bfloat16),
            jax.ShapeDtypeStruct((NPAD, WS), jnp.bfloat16),
            jax.ShapeDtypeStruct((NPAD, WS), jnp.bfloat16),
            jax.ShapeDtypeStruct((NPAD, WS), jnp.bfloat16),
        ),
        mesh=mesh,
        compiler_params=pltpu.CompilerParams(use_tc_tiling_on_sc=False),
        scratch_types=[
            pltpu.VMEM((nch, CH), jnp.int32),
            pltpu.VMEM((CH, WS), jnp.bfloat16),
            pltpu.VMEM((CH, WS), jnp.bfloat16),
            pltpu.VMEM((CH, WS), jnp.bfloat16),
            pltpu.VMEM((CH, WS), jnp.bfloat16),
            pltpu.VMEM((CH, WS), jnp.bfloat16),
            pltpu.VMEM((CH, WS), jnp.bfloat16),
            pltpu.VMEM_SHARED((NPAD, WS), jnp.bfloat16),
            pltpu.VMEM_SHARED((NPAD, WS), jnp.bfloat16),
            pltpu.SemaphoreType.DMA,
            pltpu.SemaphoreType.DMA,
        ],
    )
    return fn(feat, row2d)


def _tc_edge_body(x_ref, w1r, ew2, eb2, cw1, cb1, cw2r, o_ref):
    x = x_ref[...]
    hi = x[:, 64:80]
    radial = jnp.sum(hi * hi, axis=1, keepdims=True)
    m = _silu(x[:, 0:64] + radial * w1r[...])
    f = _silu(jnp.dot(m, ew2[...], preferred_element_type=jnp.float32) + eb2[...])
    c2 = _silu(jnp.dot(f, cw1[...], preferred_element_type=jnp.float32) + cb1[...])
    phi = jnp.sum(c2 * cw2r[...], axis=1, keepdims=True)
    e3 = (lax.broadcasted_iota(jnp.int32, (1, 16), 1) == 3).astype(jnp.float32)
    o_ref[...] = jnp.concatenate(
        [f, hi * phi + e3, jnp.zeros((BE, WS - W), jnp.float32)],
        axis=1).astype(jnp.bfloat16)


def _tc_edge(mpre, w1r, ew2, eb2, cw1, cb1, cw2r):
    epart = mpre.shape[0]
    full = lambda shape: pl.BlockSpec(shape, lambda i: (0,) * len(shape))
    return pl.pallas_call(
        _tc_edge_body,
        grid=(epart // BE,),
        in_specs=[
            pl.BlockSpec((BE, W), lambda i: (i, 0)),
            full((1, H)), full((H, H)), full((1, H)),
            full((H, H)), full((1, H)), full((1, H)),
        ],
        out_specs=pl.BlockSpec((BE, WS), lambda i: (i, 0)),
        out_shape=jax.ShapeDtypeStruct((epart, WS), jnp.bfloat16),
    )(mpre, w1r, ew2, eb2, cw1, cb1, cw2r)


def _tc_init_body(h_ref, cd_ref, win, bin_, ew1a, ew1b, eb1,
                  h1_ref, ta_ref, tb_ref):
    h1 = jnp.dot(h_ref[...], win[...], preferred_element_type=jnp.float32) + bin_[...]
    h1_ref[...] = h1
    chi = cd_ref[...]
    fa = jnp.dot(h1, ew1a[...], preferred_element_type=jnp.float32) + eb1[...]
    fb = jnp.dot(h1, ew1b[...], preferred_element_type=jnp.float32)
    ta_ref[...] = jnp.concatenate([fa, chi], axis=1)
    tb_ref[...] = jnp.concatenate([fb, -chi], axis=1)


def _tc_init(hpad, cpad16, win, bin_, ew1a, ew1b, eb1):
    full = lambda shape: pl.BlockSpec(shape, lambda i: (0,) * len(shape))
    return pl.pallas_call(
        _tc_init_body,
        grid=(NPAD // BN,),
        in_specs=[
            pl.BlockSpec((BN, D_IN), lambda i: (i, 0)),
            pl.BlockSpec((BN, 16), lambda i: (i, 0)),
            full((D_IN, H)), full((1, H)),
            full((H, H)), full((H, H)), full((1, H)),
        ],
        out_specs=[
            pl.BlockSpec((BN, H), lambda i: (i, 0)),
            pl.BlockSpec((BN, W), lambda i: (i, 0)),
            pl.BlockSpec((BN, W), lambda i: (i, 0)),
        ],
        out_shape=[
            jax.ShapeDtypeStruct((NPAD, H), jnp.float32),
            jax.ShapeDtypeStruct((NPAD, W), jnp.float32),
            jax.ShapeDtypeStruct((NPAD, W), jnp.float32),
        ],
    )(hpad, cpad16, win, bin_, ew1a, ew1b, eb1)


def _tc_node_body(h_ref, cd_ref, p0_ref, p1_ref, p2_ref, p3_ref,
                  p4_ref, p5_ref, p6_ref, p7_ref,
                  nw1a, nw1b, nb1, nw2, nb2,
                  ew1a, ew1b, eb1, h_out, cd_out, ta_ref, tb_ref):
    h = h_ref[...]
    f32 = jnp.float32
    s = (((p0_ref[...].astype(f32) + p1_ref[...].astype(f32))
          + (p2_ref[...].astype(f32) + p3_ref[...].astype(f32)))
         + ((p4_ref[...].astype(f32) + p5_ref[...].astype(f32))
            + (p6_ref[...].astype(f32) + p7_ref[...].astype(f32))))
    agg = s[:, 0:64]
    hi = s[:, 64:80]
    cnt = jnp.maximum(hi[:, 3:4], 1.0)
    chi = cd_ref[...]
    mask3 = (lax.broadcasted_iota(jnp.int32, (1, 16), 1) < 3).astype(jnp.float32)
    chi_new = chi + (hi / cnt) * mask3
    hm = _silu(jnp.dot(h, nw1a[...], preferred_element_type=jnp.float32)
               + jnp.dot(agg, nw1b[...], preferred_element_type=jnp.float32)
               + nb1[...])
    hn = h + jnp.dot(hm, nw2[...], preferred_element_type=jnp.float32) + nb2[...]
    h_out[...] = hn
    cd_out[...] = chi_new
    fa = jnp.dot(hn, ew1a[...], preferred_element_type=jnp.float32) + eb1[...]
    fb = jnp.dot(hn, ew1b[...], preferred_element_type=jnp.float32)
    ta_ref[...] = jnp.concatenate([fa, chi_new], axis=1)
    tb_ref[...] = jnp.concatenate([fb, -chi_new], axis=1)


def _tc_node(h, cd16, ps,
             nw1a, nw1b, nb1, nw2, nb2, ew1a, ew1b, eb1):
    full = lambda shape: pl.BlockSpec(shape, lambda i: (0,) * len(shape))
    return pl.pallas_call(
        _tc_node_body,
        grid=(NPAD // BN,),
        in_specs=[
            pl.BlockSpec((BN, H), lambda i: (i, 0)),
            pl.BlockSpec((BN, 16), lambda i: (i, 0)),
        ] + [pl.BlockSpec((BN, WS), lambda i: (i, 0))] * 8 + [
            full((H, H)), full((H, H)), full((1, H)), full((H, H)), full((1, H)),
            full((H, H)), full((H, H)), full((1, H)),
        ],
        out_specs=[
            pl.BlockSpec((BN, H), lambda i: (i, 0)),
            pl.BlockSpec((BN, 16), lambda i: (i, 0)),
            pl.BlockSpec((BN, W), lambda i: (i, 0)),
            pl.BlockSpec((BN, W), lambda i: (i, 0)),
        ],
        out_shape=[
            jax.ShapeDtypeStruct((NPAD, H), jnp.float32),
            jax.ShapeDtypeStruct((NPAD, 16), jnp.float32),
            jax.ShapeDtypeStruct((NPAD, W), jnp.float32),
            jax.ShapeDtypeStruct((NPAD, W), jnp.float32),
        ],
    )(h, cd16, *ps, nw1a, nw1b, nb1, nw2, nb2, ew1a, ew1b, eb1)


def _tc_final_body(h_ref, p0_ref, p1_ref, p2_ref, p3_ref,
                   p4_ref, p5_ref, p6_ref, p7_ref,
                   nw1a, nw1b, nb1, nw2, nb2,
                   wout, bout, b_ref, o_ref):
    i = pl.program_id(0)
    h = h_ref[...]
    f32 = jnp.float32
    s = (((p0_ref[...].astype(f32) + p1_ref[...].astype(f32))
          + (p2_ref[...].astype(f32) + p3_ref[...].astype(f32)))
         + ((p4_ref[...].astype(f32) + p5_ref[...].astype(f32))
            + (p6_ref[...].astype(f32) + p7_ref[...].astype(f32))))
    agg = s[:, 0:64]
    hm = _silu(jnp.dot(h, nw1a[...], preferred_element_type=jnp.float32)
               + jnp.dot(agg, nw1b[...], preferred_element_type=jnp.float32)
               + nb1[...])
    hn = h + jnp.dot(hm, nw2[...], preferred_element_type=jnp.float32) + nb2[...]
    hf = jnp.dot(hn, wout[...], preferred_element_type=jnp.float32) + bout[...]
    hext = jnp.concatenate([hf, jnp.ones((BN, 1), jnp.float32)], axis=1)
    b = b_ref[...][0]
    rows = lax.broadcasted_iota(jnp.int32, (G, BN), 0)
    oh = (rows == b).astype(jnp.float32)
    contrib = jnp.dot(oh, hext, preferred_element_type=jnp.float32)

    @pl.when(i == 0)
    def _():
        o_ref[...] = contrib

    @pl.when(i > 0)
    def _():
        o_ref[...] = o_ref[...] + contrib

    @pl.when(i == NPAD // BN - 1)
    def _():
        acc = o_ref[...]
        o_ref[...] = acc / jnp.maximum(acc[:, D_OUT:D_OUT + 1], 1.0)


def _tc_final(h, ps,
              nw1a, nw1b, nb1, nw2, nb2, wout, bout, batch3d):
    full = lambda shape: pl.BlockSpec(shape, lambda i: (0,) * len(shape))
    return pl.pallas_call(
        _tc_final_body,
        grid=(NPAD // BN,),
        in_specs=[
            pl.BlockSpec((BN, H), lambda i: (i, 0)),
        ] + [pl.BlockSpec((BN, WS), lambda i: (i, 0))] * 8 + [
            full((H, H)), full((H, H)), full((1, H)), full((H, H)), full((1, H)),
            full((H, D_OUT)), full((1, D_OUT)),
            pl.BlockSpec((1, 1, BN), lambda i: (i, 0, 0)),
        ],
        out_specs=pl.BlockSpec((G, D_OUT + 1), lambda i: (0, 0)),
        out_shape=jax.ShapeDtypeStruct((G, D_OUT + 1), jnp.float32),
    )(h, *ps, nw1a, nw1b, nb1, nw2, nb2, wout, bout, batch3d)


@jax.jit
def _run(h, x, W_in, b_in, edge_w1, edge_b1, edge_w2, edge_b2,
         coord_w1, coord_b1, coord_w2, node_w1, node_b1, node_w2, node_b2,
         W_out, b_out, edge_index, batch):
    f32 = jnp.float32
    hpad = jnp.zeros((NPAD, D_IN), f32).at[:N].set(h)
    cpad16 = jnp.zeros((NPAD, 16), f32).at[:N, :3].set(x)
    rowp = jnp.full((EPAD,), N, jnp.int32).at[:E].set(edge_index[0])
    colp = jnp.full((EPAD,), N, jnp.int32).at[:E].set(edge_index[1])
    rowp = rowp.reshape(EPAD // CH, CH)
    colp = colp.reshape(EPAD // CH, CH)
    batch3d = jnp.full((NPAD,), G + 1, jnp.int32).at[:N].set(batch)
    batch3d = batch3d.reshape(NPAD // BN, 1, BN)

    def r1(v):
        return v.reshape(1, -1)

    h1, ta, tb = _tc_init(hpad, cpad16, W_in, r1(b_in),
                          edge_w1[0, 0:H], edge_w1[0, H:2 * H], r1(edge_b1[0]))
    hcur = h1
    nch_h = NCHUNK // 2
    cb1_ = NW * nch_h
    for l in range(L):
        ew = (r1(edge_w1[l, 2 * H]), edge_w2[l], r1(edge_b2[l]),
              coord_w1[l], r1(coord_b1[l]), r1(coord_w2[l, :, 0]))
        m0 = _sc_gather(ta, tb, rowp, colp, nch_h, 0)
        m1 = _sc_gather(ta, tb, rowp, colp, nch_h, cb1_)
        f0 = _tc_edge(m0, *ew)
        f1 = _tc_edge(m1, *ew)
        ps = (_sc_scatter(f0, rowp, nch_h, 0)
              + _sc_scatter(f1, rowp, nch_h, cb1_))
        if l < L - 1:
            hcur, cpad16_n, ta, tb = _tc_node(
                hcur, cpad16, ps,
                node_w1[l, 0:H], node_w1[l, H:2 * H], r1(node_b1[l]),
                node_w2[l], r1(node_b2[l]),
                edge_w1[l + 1, 0:H], edge_w1[l + 1, H:2 * H], r1(edge_b1[l + 1]))
            cpad16 = cpad16_n
        else:
            out = _tc_final(hcur, ps,
                            node_w1[l, 0:H], node_w1[l, H:2 * H], r1(node_b1[l]),
                            node_w2[l], r1(node_b2[l]),
                            W_out, r1(b_out), batch3d)
    return out[:, :D_OUT]


def kernel(h, x, W_in, b_in, edge_w1, edge_b1, edge_w2, edge_b2,
           coord_w1, coord_b1, coord_w2, node_w1, node_b1, node_w2, node_b2,
           W_out, b_out, edge_index, batch):
    return _run(h, x, W_in, b_in, edge_w1, edge_b1, edge_w2, edge_b2,
                coord_w1, coord_b1, coord_w2, node_w1, node_b1, node_w2,
                node_b2, W_out, b_out, edge_index, batch)


# speedup vs baseline: 1.2492x; 1.1059x over previous
import functools

import jax
import jax.numpy as jnp
from jax import lax
from jax.experimental import pallas as pl
from jax.experimental.pallas import tpu as pltpu
from jax.experimental.pallas import tpu_sc as plsc

N = 10000
E = 320000
D_IN = 128
H = 64
D_OUT = 128
L = 4
G = 16

W = 80
NPAD = 10240
NT = 10016
NW = 32
EW = 10240
EPAD = NW * EW
CH = 128
NCHUNK = EW // CH
BN = 2048
BE = 2048


def _silu(x):
    return x * jax.lax.logistic(x)


def _make_sc_gather_body(nch, cbase):
    ew = nch * CH

    def body(ta, tb, row2d, col2d, out, idxr, idxc,
             a0, a1, a2, a3, b0, b1, b2, b3, sem_g, sem_s):
        c = lax.axis_index("c")
        s = lax.axis_index("s")
        wid = s * 2 + c
        ab = (a0, a1, a2, a3)
        bb = (b0, b1, b2, b3)

        pltpu.sync_copy(row2d.at[pl.ds(cbase + wid * nch, nch)], idxr)
        pltpu.sync_copy(col2d.at[pl.ds(cbase + wid * nch, nch)], idxc)

        def fire(k, p):
            pltpu.async_copy(ta.at[idxr.at[k]], ab[p], sem_g)
            pltpu.async_copy(tb.at[idxc.at[k]], bb[p], sem_g)

        fire(0, 0)
        fire(1, 1)

        def addp(ap, bp):
            @plsc.parallel_loop(0, CH, unroll=4)
            def _(g):
                for i in range(5):
                    sl = pl.ds(i * 16, 16)
                    ap[g, sl] = ap[g, sl] + bp[g, sl]

        def quad(j, carry):
            for q in range(4):
                k = 4 * j + q
                base = wid * ew + k * CH
                dummy = out.at[pl.ds(base, CH)]
                pltpu.make_async_copy(dummy, a0, sem_g).wait()
                pltpu.make_async_copy(dummy, b0, sem_g).wait()

                @pl.when(k >= 2)
                def _():
                    pltpu.make_async_copy(a0, dummy, sem_s).wait()

                @pl.when(k + 2 < nch)
                def _():
                    fire(k + 2, (q + 2) % 4)

                addp(ab[q], bb[q])
                pltpu.async_copy(ab[q], out.at[pl.ds(base, CH)], sem_s)
            return carry

        lax.fori_loop(0, nch // 4, quad, 0)
        dummy = out.at[pl.ds(wid * ew, CH)]
        pltpu.make_async_copy(a0, dummy, sem_s).wait()
        pltpu.make_async_copy(a0, dummy, sem_s).wait()

    return body


def _sc_gather(ta, tb, row2d, col2d, nch, cbase):
    mesh = plsc.VectorSubcoreMesh(core_axis_name="c", subcore_axis_name="s")
    fn = pl.kernel(
        _make_sc_gather_body(nch, cbase),
        out_type=jax.ShapeDtypeStruct((NW * nch * CH, W), jnp.float32),
        mesh=mesh,
        compiler_params=pltpu.CompilerParams(use_tc_tiling_on_sc=False),
        scratch_types=[
            pltpu.VMEM((nch, CH), jnp.int32),
            pltpu.VMEM((nch, CH), jnp.int32),
            pltpu.VMEM((CH, W), jnp.float32),
            pltpu.VMEM((CH, W), jnp.float32),
            pltpu.VMEM((CH, W), jnp.float32),
            pltpu.VMEM((CH, W), jnp.float32),
            pltpu.VMEM((CH, W), jnp.float32),
            pltpu.VMEM((CH, W), jnp.float32),
            pltpu.VMEM((CH, W), jnp.float32),
            pltpu.VMEM((CH, W), jnp.float32),
            pltpu.SemaphoreType.DMA,
            pltpu.SemaphoreType.DMA,
        ],
    )
    return fn(ta, tb, row2d, col2d)


def _make_sc_scatter_body(nch, cbase):
    ew = nch * CH

    def body(feat, row2d, out0, out1, idx,
             f0, f1, f2, f3, f4, f5, acc, sem_l, sem_sc):
        c = lax.axis_index("c")
        s = lax.axis_index("s")
        wid = s * 2 + c
        fb = (f0, f1, f2, f3, f4, f5)

        pltpu.sync_copy(row2d.at[pl.ds(cbase + wid * nch, nch)], idx)

        zv = jnp.zeros((16,), jnp.float32)

        def zrow(g, cc):
            for i in range(5):
                f0[g, pl.ds(i * 16, 16)] = zv
            return cc

        lax.fori_loop(0, CH, zrow, 0)

        def zcp(j, cc):
            pltpu.sync_copy(f0, acc.at[pl.ds(s * (NPAD // 16) + j * CH, CH)])
            return cc

        lax.fori_loop(0, (NPAD // 16) // CH, zcp, 0)
        plsc.subcore_barrier()

        def fire(k, p):
            pltpu.async_copy(feat.at[pl.ds(wid * ew + k * CH, CH)], fb[p],
                             sem_l)

        fire(0, 0)
        fire(1, 1)
        fire(2, 2)

        def step(k, q):
            dummy = feat.at[pl.ds(wid * ew + k * CH, CH)]
            pltpu.make_async_copy(dummy, f0, sem_l).wait()

            @pl.when(k >= 3)
            def _():
                pltpu.make_async_copy(dummy, f0, sem_sc).wait()

            @pl.when(k + 3 < nch)
            def _():
                fire(k + 3, (q + 3) % 6)

            pltpu.async_copy(fb[q], acc.at[idx.at[k]], sem_sc, add=True)

        def hexa(j, cc):
            for q in range(6):
                step(6 * j + q, q)
            return cc

        lax.fori_loop(0, nch // 6, hexa, 0)
        for q in range(nch - (nch // 6) * 6):
            k = (nch // 6) * 6 + q
            step(k, k % 6)
        dummy = feat.at[pl.ds(wid * ew, CH)]
        for _ in range(3):
            pltpu.make_async_copy(dummy, f0, sem_sc).wait()
        plsc.subcore_barrier()

        def cpout(j, cc):
            off = s * (NPAD // 16) + j * CH

            @pl.when(c == 0)
            def _():
                pltpu.sync_copy(acc.at[pl.ds(off, CH)],
                                out0.at[pl.ds(off, CH)])

            @pl.when(c == 1)
            def _():
                pltpu.sync_copy(acc.at[pl.ds(off, CH)],
                                out1.at[pl.ds(off, CH)])

            return cc

        lax.fori_loop(0, (NPAD // 16) // CH, cpout, 0)

    return body


def _sc_scatter(feat, row2d, nch, cbase):
    mesh = plsc.VectorSubcoreMesh(core_axis_name="c", subcore_axis_name="s")
    fn = pl.kernel(
        _make_sc_scatter_body(nch, cbase),
        out_type=(
            jax.ShapeDtypeStruct((NPAD, W), jnp.float32),
            jax.ShapeDtypeStruct((NPAD, W), jnp.float32),
        ),
        mesh=mesh,
        compiler_params=pltpu.CompilerParams(use_tc_tiling_on_sc=False),
        scratch_types=[
            pltpu.VMEM((nch, CH), jnp.int32),
            pltpu.VMEM((CH, W), jnp.float32),
            pltpu.VMEM((CH, W), jnp.float32),
            pltpu.VMEM((CH, W), jnp.float32),
            pltpu.VMEM((CH, W), jnp.float32),
            pltpu.VMEM((CH, W), jnp.float32),
            pltpu.VMEM((CH, W), jnp.float32),
            pltpu.VMEM_SHARED((NPAD, W), jnp.float32),
            pltpu.SemaphoreType.DMA,
            pltpu.SemaphoreType.DMA,
        ],
    )
    return fn(feat, row2d)


def _tc_edge_body(x_ref, w1r, ew2, eb2, cw1, cb1, cw2r, o_ref):
    x = x_ref[...]
    hi = x[:, 64:80]
    radial = jnp.sum(hi * hi, axis=1, keepdims=True)
    m = _silu(x[:, 0:64] + radial * w1r[...])
    f = _silu(jnp.dot(m, ew2[...], preferred_element_type=jnp.float32) + eb2[...])
    c2 = _silu(jnp.dot(f, cw1[...], preferred_element_type=jnp.float32) + cb1[...])
    phi = jnp.sum(c2 * cw2r[...], axis=1, keepdims=True)
    e3 = (lax.broadcasted_iota(jnp.int32, (1, 16), 1) == 3).astype(jnp.float32)
    o_ref[...] = jnp.concatenate([f, hi * phi + e3], axis=1)


def _tc_edge(mpre, w1r, ew2, eb2, cw1, cb1, cw2r):
    epart = mpre.shape[0]
    full = lambda shape: pl.BlockSpec(shape, lambda i: (0,) * len(shape))
    return pl.pallas_call(
        _tc_edge_body,
        grid=(epart // BE,),
        in_specs=[
            pl.BlockSpec((BE, W), lambda i: (i, 0)),
            full((1, H)), full((H, H)), full((1, H)),
            full((H, H)), full((1, H)), full((1, H)),
        ],
        out_specs=pl.BlockSpec((BE, W), lambda i: (i, 0)),
        out_shape=jax.ShapeDtypeStruct((epart, W), jnp.float32),
    )(mpre, w1r, ew2, eb2, cw1, cb1, cw2r)


def _tc_init_body(h_ref, cd_ref, win, bin_, ew1a, ew1b, eb1,
                  h1_ref, ta_ref, tb_ref):
    h1 = jnp.dot(h_ref[...], win[...], preferred_element_type=jnp.float32) + bin_[...]
    h1_ref[...] = h1
    chi = cd_ref[...]
    fa = jnp.dot(h1, ew1a[...], preferred_element_type=jnp.float32) + eb1[...]
    fb = jnp.dot(h1, ew1b[...], preferred_element_type=jnp.float32)
    ta_ref[...] = jnp.concatenate([fa, chi], axis=1)
    tb_ref[...] = jnp.concatenate([fb, -chi], axis=1)


def _tc_init(hpad, cpad16, win, bin_, ew1a, ew1b, eb1):
    full = lambda shape: pl.BlockSpec(shape, lambda i: (0,) * len(shape))
    return pl.pallas_call(
        _tc_init_body,
        grid=(NPAD // BN,),
        in_specs=[
            pl.BlockSpec((BN, D_IN), lambda i: (i, 0)),
            pl.BlockSpec((BN, 16), lambda i: (i, 0)),
            full((D_IN, H)), full((1, H)),
            full((H, H)), full((H, H)), full((1, H)),
        ],
        out_specs=[
            pl.BlockSpec((BN, H), lambda i: (i, 0)),
            pl.BlockSpec((BN, W), lambda i: (i, 0)),
            pl.BlockSpec((BN, W), lambda i: (i, 0)),
        ],
        out_shape=[
            jax.ShapeDtypeStruct((NPAD, H), jnp.float32),
            jax.ShapeDtypeStruct((NPAD, W), jnp.float32),
            jax.ShapeDtypeStruct((NPAD, W), jnp.float32),
        ],
    )(hpad, cpad16, win, bin_, ew1a, ew1b, eb1)


def _tc_node_body(h_ref, cd_ref, p0_ref, p1_ref, p2_ref, p3_ref,
                  p4_ref, p5_ref, p6_ref, p7_ref,
                  nw1a, nw1b, nb1, nw2, nb2,
                  ew1a, ew1b, eb1, h_out, cd_out, ta_ref, tb_ref):
    h = h_ref[...]
    s = (((p0_ref[...] + p1_ref[...]) + (p2_ref[...] + p3_ref[...]))
         + ((p4_ref[...] + p5_ref[...]) + (p6_ref[...] + p7_ref[...])))
    agg = s[:, 0:64]
    hi = s[:, 64:80]
    cnt = jnp.maximum(hi[:, 3:4], 1.0)
    chi = cd_ref[...]
    mask3 = (lax.broadcasted_iota(jnp.int32, (1, 16), 1) < 3).astype(jnp.float32)
    chi_new = chi + (hi / cnt) * mask3
    hm = _silu(jnp.dot(h, nw1a[...], preferred_element_type=jnp.float32)
               + jnp.dot(agg, nw1b[...], preferred_element_type=jnp.float32)
               + nb1[...])
    hn = h + jnp.dot(hm, nw2[...], preferred_element_type=jnp.float32) + nb2[...]
    h_out[...] = hn
    cd_out[...] = chi_new
    fa = jnp.dot(hn, ew1a[...], preferred_element_type=jnp.float32) + eb1[...]
    fb = jnp.dot(hn, ew1b[...], preferred_element_type=jnp.float32)
    ta_ref[...] = jnp.concatenate([fa, chi_new], axis=1)
    tb_ref[...] = jnp.concatenate([fb, -chi_new], axis=1)


def _tc_node(h, cd16, ps,
             nw1a, nw1b, nb1, nw2, nb2, ew1a, ew1b, eb1):
    full = lambda shape: pl.BlockSpec(shape, lambda i: (0,) * len(shape))
    return pl.pallas_call(
        _tc_node_body,
        grid=(NPAD // BN,),
        in_specs=[
            pl.BlockSpec((BN, H), lambda i: (i, 0)),
            pl.BlockSpec((BN, 16), lambda i: (i, 0)),
        ] + [pl.BlockSpec((BN, W), lambda i: (i, 0))] * 8 + [
            full((H, H)), full((H, H)), full((1, H)), full((H, H)), full((1, H)),
            full((H, H)), full((H, H)), full((1, H)),
        ],
        out_specs=[
            pl.BlockSpec((BN, H), lambda i: (i, 0)),
            pl.BlockSpec((BN, 16), lambda i: (i, 0)),
            pl.BlockSpec((BN, W), lambda i: (i, 0)),
            pl.BlockSpec((BN, W), lambda i: (i, 0)),
        ],
        out_shape=[
            jax.ShapeDtypeStruct((NPAD, H), jnp.float32),
            jax.ShapeDtypeStruct((NPAD, 16), jnp.float32),
            jax.ShapeDtypeStruct((NPAD, W), jnp.float32),
            jax.ShapeDtypeStruct((NPAD, W), jnp.float32),
        ],
    )(h, cd16, *ps, nw1a, nw1b, nb1, nw2, nb2, ew1a, ew1b, eb1)


def _tc_final_body(h_ref, p0_ref, p1_ref, p2_ref, p3_ref,
                   p4_ref, p5_ref, p6_ref, p7_ref,
                   nw1a, nw1b, nb1, nw2, nb2,
                   wout, bout, b_ref, o_ref):
    i = pl.program_id(0)
    h = h_ref[...]
    s = (((p0_ref[...] + p1_ref[...]) + (p2_ref[...] + p3_ref[...]))
         + ((p4_ref[...] + p5_ref[...]) + (p6_ref[...] + p7_ref[...])))
    agg = s[:, 0:64]
    hm = _silu(jnp.dot(h, nw1a[...], preferred_element_type=jnp.float32)
               + jnp.dot(agg, nw1b[...], preferred_element_type=jnp.float32)
               + nb1[...])
    hn = h + jnp.dot(hm, nw2[...], preferred_element_type=jnp.float32) + nb2[...]
    hf = jnp.dot(hn, wout[...], preferred_element_type=jnp.float32) + bout[...]
    hext = jnp.concatenate([hf, jnp.ones((BN, 1), jnp.float32)], axis=1)
    b = b_ref[...][0]
    rows = lax.broadcasted_iota(jnp.int32, (G, BN), 0)
    oh = (rows == b).astype(jnp.float32)
    contrib = jnp.dot(oh, hext, preferred_element_type=jnp.float32)

    @pl.when(i == 0)
    def _():
        o_ref[...] = contrib

    @pl.when(i > 0)
    def _():
        o_ref[...] = o_ref[...] + contrib

    @pl.when(i == NPAD // BN - 1)
    def _():
        acc = o_ref[...]
        o_ref[...] = acc / jnp.maximum(acc[:, D_OUT:D_OUT + 1], 1.0)


def _tc_final(h, ps,
              nw1a, nw1b, nb1, nw2, nb2, wout, bout, batch3d):
    full = lambda shape: pl.BlockSpec(shape, lambda i: (0,) * len(shape))
    return pl.pallas_call(
        _tc_final_body,
        grid=(NPAD // BN,),
        in_specs=[
            pl.BlockSpec((BN, H), lambda i: (i, 0)),
        ] + [pl.BlockSpec((BN, W), lambda i: (i, 0))] * 8 + [
            full((H, H)), full((H, H)), full((1, H)), full((H, H)), full((1, H)),
            full((H, D_OUT)), full((1, D_OUT)),
            pl.BlockSpec((1, 1, BN), lambda i: (i, 0, 0)),
        ],
        out_specs=pl.BlockSpec((G, D_OUT + 1), lambda i: (0, 0)),
        out_shape=jax.ShapeDtypeStruct((G, D_OUT + 1), jnp.float32),
    )(h, *ps, nw1a, nw1b, nb1, nw2, nb2, wout, bout, batch3d)


@jax.jit
def _run(h, x, W_in, b_in, edge_w1, edge_b1, edge_w2, edge_b2,
         coord_w1, coord_b1, coord_w2, node_w1, node_b1, node_w2, node_b2,
         W_out, b_out, edge_index, batch):
    f32 = jnp.float32
    hpad = jnp.zeros((NPAD, D_IN), f32).at[:N].set(h)
    cpad16 = jnp.zeros((NPAD, 16), f32).at[:N, :3].set(x)
    rowp = jnp.full((EPAD,), N, jnp.int32).at[:E].set(edge_index[0])
    colp = jnp.full((EPAD,), N, jnp.int32).at[:E].set(edge_index[1])
    rowp = rowp.reshape(EPAD // CH, CH)
    colp = colp.reshape(EPAD // CH, CH)
    batch3d = jnp.full((NPAD,), G + 1, jnp.int32).at[:N].set(batch)
    batch3d = batch3d.reshape(NPAD // BN, 1, BN)

    def r1(v):
        return v.reshape(1, -1)

    h1, ta, tb = _tc_init(hpad, cpad16, W_in, r1(b_in),
                          edge_w1[0, 0:H], edge_w1[0, H:2 * H], r1(edge_b1[0]))
    hcur = h1
    nq = NCHUNK // 4
    for l in range(L):
        ew = (r1(edge_w1[l, 2 * H]), edge_w2[l], r1(edge_b2[l]),
              coord_w1[l], r1(coord_b1[l]), r1(coord_w2[l, :, 0]))
        ps = []
        fs = []
        for qq in range(4):
            mq = _sc_gather(ta, tb, rowp, colp, nq, qq * NW * nq)
            fs.append(_tc_edge(mq, *ew))
        for qq in range(4):
            ps.extend(_sc_scatter(fs[qq], rowp, nq, qq * NW * nq))
        if l < L - 1:
            hcur, cpad16_n, ta, tb = _tc_node(
                hcur, cpad16, ps,
                node_w1[l, 0:H], node_w1[l, H:2 * H], r1(node_b1[l]),
                node_w2[l], r1(node_b2[l]),
                edge_w1[l + 1, 0:H], edge_w1[l + 1, H:2 * H], r1(edge_b1[l + 1]))
            cpad16 = cpad16_n
        else:
            out = _tc_final(hcur, ps,
                            node_w1[l, 0:H], node_w1[l, H:2 * H], r1(node_b1[l]),
                            node_w2[l], r1(node_b2[l]),
                            W_out, r1(b_out), batch3d)
    return out[:, :D_OUT]


def kernel(h, x, W_in, b_in, edge_w1, edge_b1, edge_w2, edge_b2,
           coord_w1, coord_b1, coord_w2, node_w1, node_b1, node_w2, node_b2,
           W_out, b_out, edge_index, batch):
    return _run(h, x, W_in, b_in, edge_w1, edge_b1, edge_w2, edge_b2,
                coord_w1, coord_b1, coord_w2, node_w1, node_b1, node_w2,
                node_b2, W_out, b_out, edge_index, batch)
